# baseline (speedup 1.0000x reference)
import functools

import jax
import jax.numpy as jnp
from jax import lax
from jax.experimental import pallas as pl
from jax.experimental.pallas import tpu as pltpu
from jax.experimental.pallas import tpu_sc as plsc

NC = 2
NS = 16
NW = NC * NS
L = 16

N_ROWS = 4096
D = 512
DPT = D // NW
N_TRIP = 16384
CH = 2048
N_CHUNKS = N_TRIP // CH
GRP = CH // L


def _sc_partials_body(xt_hbm, ti_hbm, tj_hbm, tk_hbm, out_hbm,
                      tab, idx_i, idx_j, idx_k, oij_v, oik_v):
    cid = lax.axis_index("c")
    sid = lax.axis_index("s")
    wid = sid * NC + cid

    pltpu.sync_copy(xt_hbm.at[pl.ds(wid * DPT, DPT)], tab)

    for c in range(N_CHUNKS):
        base = c * CH
        pltpu.sync_copy(ti_hbm.at[pl.ds(base, CH)], idx_i)
        pltpu.sync_copy(tj_hbm.at[pl.ds(base, CH)], idx_j)
        pltpu.sync_copy(tk_hbm.at[pl.ds(base, CH)], idx_k)

        def grp_body(g, carry):
            off = g * L
            sl = pl.ds(off, L)
            iv = idx_i[sl]
            jv = idx_j[sl]
            kv = idx_k[sl]
            zero = jnp.zeros((L,), jnp.float32)
            a_ii = a_jj = a_kk = a_ij = a_ik = zero
            for d in range(DPT):
                dv = jnp.full((L,), d, jnp.int32)
                gi = plsc.load_gather(tab, [dv, iv])
                gj = plsc.load_gather(tab, [dv, jv])
                gk = plsc.load_gather(tab, [dv, kv])
                a_ii = a_ii + gi * gi
                a_jj = a_jj + gj * gj
                a_kk = a_kk + gk * gk
                a_ij = a_ij + gi * gj
                a_ik = a_ik + gi * gk
            oij_v[sl] = a_ii + a_jj - (a_ij + a_ij)
            oik_v[sl] = a_ii + a_kk - (a_ik + a_ik)
            return carry

        lax.fori_loop(0, GRP, grp_body, 0)
        pltpu.sync_copy(oij_v, out_hbm.at[0, wid, pl.ds(base, CH)])
        pltpu.sync_copy(oik_v, out_hbm.at[1, wid, pl.ds(base, CH)])


_sc_partials = functools.partial(
    pl.kernel,
    out_type=jax.ShapeDtypeStruct((2, NW, N_TRIP), jnp.float32),
    mesh=plsc.VectorSubcoreMesh(
        core_axis_name="c", subcore_axis_name="s",
        num_cores=NC, num_subcores=NS),
    compiler_params=pltpu.CompilerParams(needs_layout_passes=False),
    scratch_types=[
        pltpu.VMEM((DPT, N_ROWS), jnp.float32),
        pltpu.VMEM((CH,), jnp.int32),
        pltpu.VMEM((CH,), jnp.int32),
        pltpu.VMEM((CH,), jnp.int32),
        pltpu.VMEM((CH,), jnp.float32),
        pltpu.VMEM((CH,), jnp.float32),
    ],
)(_sc_partials_body)


def _tc_loss_body(p_ref, o_ref):
    dij = jnp.maximum(jnp.sum(p_ref[0], axis=0), 0.0)
    dik = jnp.maximum(jnp.sum(p_ref[1], axis=0), 0.0)
    per = jnp.log(1.0 + jnp.exp(dij - dik))
    o_ref[...] = jnp.reshape(jnp.sum(per) / float(N_TRIP), (1, 1))


def kernel(x, triplets):
    t32 = triplets.astype(jnp.int32)
    ti = t32[:, 0]
    tj = t32[:, 1]
    tk = t32[:, 2]
    xt = x.T
    partials = _sc_partials(xt, ti, tj, tk)
    loss = pl.pallas_call(
        _tc_loss_body,
        out_shape=jax.ShapeDtypeStruct((1, 1), jnp.float32),
    )(partials.reshape(2, NW, 128, 128))
    return loss.reshape(1)


# speedup vs baseline: 1.0047x; 1.0047x over previous
import functools

import jax
import jax.numpy as jnp
from jax import lax
from jax.experimental import pallas as pl
from jax.experimental.pallas import tpu as pltpu
from jax.experimental.pallas import tpu_sc as plsc

NC = 2
NS = 16
NW = NC * NS
L = 16

N_ROWS = 4096
D = 512
N_TRIP = 16384
TPT = N_TRIP // NW
GRP = TPT // L
BM = 512
NB = N_ROWS // BM
IDX_CH = 128


def _mm_body(x_ref, g_ref, n_ref):
    i = pl.program_id(0)
    j = pl.program_id(1)
    xi = x_ref[pl.ds(i * BM, BM), :]
    xj = x_ref[pl.ds(j * BM, BM), :]
    g_ref[...] = lax.dot_general(
        xi.astype(jnp.bfloat16), xj.astype(jnp.bfloat16),
        (((1,), (1,)), ((), ())),
        preferred_element_type=jnp.float32)

    @pl.when(j == 0)
    def _():
        n_ref[...] = jnp.sum(xi * xi, axis=1, keepdims=True)


def _matmul_norms(x):
    return pl.pallas_call(
        _mm_body,
        grid=(NB, NB),
        in_specs=[pl.BlockSpec((N_ROWS, D), lambda i, j: (0, 0))],
        out_specs=[
            pl.BlockSpec((BM, BM), lambda i, j: (i, j)),
            pl.BlockSpec((BM, 1), lambda i, j: (i, 0)),
        ],
        out_shape=[
            jax.ShapeDtypeStruct((N_ROWS, N_ROWS), jnp.float32),
            jax.ShapeDtypeStruct((N_ROWS, 1), jnp.float32),
        ],
    )(x)


def _sc_diff_body(g_hbm, n_hbm, ti_hbm, tj_hbm, tk_hbm, out_hbm,
                  n_v, iv_v, jv_v, kv_v, fij_v, fik_v,
                  gij_v, gik_v, diff_v, sem):
    cid = lax.axis_index("c")
    sid = lax.axis_index("s")
    wid = sid * NC + cid
    base = wid * TPT

    pltpu.sync_copy(n_hbm, n_v)
    pltpu.sync_copy(ti_hbm.at[pl.ds(base, TPT)], iv_v)
    pltpu.sync_copy(tj_hbm.at[pl.ds(base, TPT)], jv_v)
    pltpu.sync_copy(tk_hbm.at[pl.ds(base, TPT)], kv_v)

    def build_body(g, carry):
        sl = pl.ds(g * L, L)
        iv = iv_v[sl]
        rowbase = lax.shift_left(iv, 12)
        fij_v[sl] = rowbase + jv_v[sl]
        fik_v[sl] = rowbase + kv_v[sl]
        return carry

    lax.fori_loop(0, GRP, build_body, 0)

    copies = []
    for c in range(TPT // IDX_CH):
        sl = pl.ds(c * IDX_CH, IDX_CH)
        copies.append(pltpu.async_copy(
            g_hbm.at[fij_v.at[sl]], gij_v.at[sl], sem))
        copies.append(pltpu.async_copy(
            g_hbm.at[fik_v.at[sl]], gik_v.at[sl], sem))
    for cp in copies:
        cp.wait()

    def diff_body(g, carry):
        sl = pl.ds(g * L, L)
        ni = plsc.load_gather(n_v, [iv_v[sl]])
        nj = plsc.load_gather(n_v, [jv_v[sl]])
        nk = plsc.load_gather(n_v, [kv_v[sl]])
        gij = gij_v[sl]
        gik = gik_v[sl]
        dij = jnp.maximum(ni + nj - (gij + gij), 0.0)
        dik = jnp.maximum(ni + nk - (gik + gik), 0.0)
        diff_v[sl] = dij - dik
        return carry

    lax.fori_loop(0, GRP, diff_body, 0)
    pltpu.sync_copy(diff_v, out_hbm.at[pl.ds(base, TPT)])


_sc_diffs = functools.partial(
    pl.kernel,
    out_type=jax.ShapeDtypeStruct((N_TRIP,), jnp.float32),
    mesh=plsc.VectorSubcoreMesh(
        core_axis_name="c", subcore_axis_name="s",
        num_cores=NC, num_subcores=NS),
    compiler_params=pltpu.CompilerParams(needs_layout_passes=False),
    scratch_types=[
        pltpu.VMEM((N_ROWS,), jnp.float32),
        pltpu.VMEM((TPT,), jnp.int32),
        pltpu.VMEM((TPT,), jnp.int32),
        pltpu.VMEM((TPT,), jnp.int32),
        pltpu.VMEM((TPT,), jnp.int32),
        pltpu.VMEM((TPT,), jnp.int32),
        pltpu.VMEM((TPT,), jnp.float32),
        pltpu.VMEM((TPT,), jnp.float32),
        pltpu.VMEM((TPT,), jnp.float32),
        pltpu.SemaphoreType.DMA,
    ],
)(_sc_diff_body)


def _tc_loss_body(d_ref, o_ref):
    per = jnp.log(1.0 + jnp.exp(d_ref[...]))
    o_ref[...] = jnp.reshape(jnp.sum(per) / float(N_TRIP), (1, 1))


def kernel(x, triplets):
    t32 = triplets.astype(jnp.int32)
    ti = t32[:, 0]
    tj = t32[:, 1]
    tk = t32[:, 2]
    g, n = _matmul_norms(x)
    diffs = _sc_diffs(g.reshape(N_ROWS * N_ROWS), n.reshape(N_ROWS),
                      ti, tj, tk)
    loss = pl.pallas_call(
        _tc_loss_body,
        out_shape=jax.ShapeDtypeStruct((1, 1), jnp.float32),
    )(diffs.reshape(128, 128))
    return loss.reshape(1)


# speedup vs baseline: 1.1528x; 1.1474x over previous
import functools

import jax
import jax.numpy as jnp
from jax import lax
from jax.experimental import pallas as pl
from jax.experimental.pallas import tpu as pltpu
from jax.experimental.pallas import tpu_sc as plsc

NC = 2
NS = 16
NW = NC * NS
L = 16

N_ROWS = 4096
D = 512
N_TRIP = 16384
TPT = N_TRIP // NW
GRP = TPT // L
BM = 512
NB = N_ROWS // BM
IDX_CH = 128


def _mm_body(xb_ref, g_ref, n_ref):
    i = pl.program_id(0)
    xi = xb_ref[pl.ds(i * BM, BM), :]
    g_ref[...] = lax.dot_general(
        xi, xb_ref[...],
        (((1,), (1,)), ((), ())),
        preferred_element_type=jnp.float32)
    xif = xi.astype(jnp.float32)
    n_ref[...] = jnp.sum(xif * xif, axis=1, keepdims=True)


def _matmul_norms(xb):
    return pl.pallas_call(
        _mm_body,
        grid=(NB,),
        in_specs=[pl.BlockSpec((N_ROWS, D), lambda i: (0, 0))],
        out_specs=[
            pl.BlockSpec((BM, N_ROWS), lambda i: (i, 0)),
            pl.BlockSpec((BM, 1), lambda i: (i, 0)),
        ],
        out_shape=[
            jax.ShapeDtypeStruct((N_ROWS, N_ROWS), jnp.float32),
            jax.ShapeDtypeStruct((N_ROWS, 1), jnp.float32),
        ],
    )(xb)


def _sc_diff_body(g_hbm, n_hbm, t_hbm, out_hbm,
                  n_v, tv_v, iv_v, jv_v, kv_v, fij_v, fik_v,
                  gij_v, gik_v, diff_v, sem):
    cid = lax.axis_index("c")
    sid = lax.axis_index("s")
    wid = sid * NC + cid
    base = wid * TPT

    pltpu.sync_copy(n_hbm, n_v)
    pltpu.sync_copy(t_hbm.at[pl.ds(base * 3, TPT * 3)], tv_v)

    lanes3 = lax.iota(jnp.int32, L) * 3

    def build_body(g, carry):
        sl = pl.ds(g * L, L)
        b = lanes3 + g * (3 * L)
        iv = plsc.load_gather(tv_v, [b])
        jv = plsc.load_gather(tv_v, [b + 1])
        kv = plsc.load_gather(tv_v, [b + 2])
        iv_v[sl] = iv
        jv_v[sl] = jv
        kv_v[sl] = kv
        rowbase = lax.shift_left(iv, 12)
        fij_v[sl] = rowbase + jv
        fik_v[sl] = rowbase + kv
        return carry

    lax.fori_loop(0, GRP, build_body, 0)

    copies = []
    for c in range(TPT // IDX_CH):
        sl = pl.ds(c * IDX_CH, IDX_CH)
        copies.append(pltpu.async_copy(
            g_hbm.at[fij_v.at[sl]], gij_v.at[sl], sem))
        copies.append(pltpu.async_copy(
            g_hbm.at[fik_v.at[sl]], gik_v.at[sl], sem))
    for cp in copies:
        cp.wait()

    def diff_body(g, carry):
        sl = pl.ds(g * L, L)
        ni = plsc.load_gather(n_v, [iv_v[sl]])
        nj = plsc.load_gather(n_v, [jv_v[sl]])
        nk = plsc.load_gather(n_v, [kv_v[sl]])
        gij = gij_v[sl]
        gik = gik_v[sl]
        dij = jnp.maximum(ni + nj - (gij + gij), 0.0)
        dik = jnp.maximum(ni + nk - (gik + gik), 0.0)
        diff_v[sl] = dij - dik
        return carry

    lax.fori_loop(0, GRP, diff_body, 0)
    pltpu.sync_copy(diff_v, out_hbm.at[pl.ds(base, TPT)])


_sc_diffs = functools.partial(
    pl.kernel,
    out_type=jax.ShapeDtypeStruct((N_TRIP,), jnp.float32),
    mesh=plsc.VectorSubcoreMesh(
        core_axis_name="c", subcore_axis_name="s",
        num_cores=NC, num_subcores=NS),
    compiler_params=pltpu.CompilerParams(needs_layout_passes=False),
    scratch_types=[
        pltpu.VMEM((N_ROWS,), jnp.float32),
        pltpu.VMEM((TPT * 3,), jnp.int32),
        pltpu.VMEM((TPT,), jnp.int32),
        pltpu.VMEM((TPT,), jnp.int32),
        pltpu.VMEM((TPT,), jnp.int32),
        pltpu.VMEM((TPT,), jnp.int32),
        pltpu.VMEM((TPT,), jnp.int32),
        pltpu.VMEM((TPT,), jnp.float32),
        pltpu.VMEM((TPT,), jnp.float32),
        pltpu.VMEM((TPT,), jnp.float32),
        pltpu.SemaphoreType.DMA,
    ],
)(_sc_diff_body)


def _tc_loss_body(d_ref, o_ref):
    per = jnp.log(1.0 + jnp.exp(d_ref[...]))
    o_ref[...] = jnp.reshape(jnp.sum(per) / float(N_TRIP), (1, 1))


def kernel(x, triplets):
    t_flat = triplets.astype(jnp.int32).reshape(N_TRIP * 3)
    g, n = _matmul_norms(x.astype(jnp.bfloat16))
    diffs = _sc_diffs(g.reshape(N_ROWS * N_ROWS), n.reshape(N_ROWS),
                      t_flat)
    loss = pl.pallas_call(
        _tc_loss_body,
        out_shape=jax.ShapeDtypeStruct((1, 1), jnp.float32),
    )(diffs.reshape(128, 128))
    return loss.reshape(1)


# speedup vs baseline: 1.1953x; 1.0369x over previous
import functools

import jax
import jax.numpy as jnp
from jax import lax
from jax.experimental import pallas as pl
from jax.experimental.pallas import tpu as pltpu
from jax.experimental.pallas import tpu_sc as plsc

NC = 2
NS = 16
NW = NC * NS
L = 16

N_ROWS = 4096
D = 512
N_TRIP = 16384
TPT = N_TRIP // NW
GRP = TPT // L
BM = 512
NB = N_ROWS // BM
IDX_CH = 128


def _mm_body(x_ref, g_ref, n_ref, xb_scr):
    i = pl.program_id(0)

    @pl.when(i == 0)
    def _():
        xb_scr[...] = x_ref[...].astype(jnp.bfloat16)

    xi = x_ref[pl.ds(i * BM, BM), :]
    g_ref[...] = lax.dot_general(
        xb_scr[pl.ds(i * BM, BM), :], xb_scr[...],
        (((1,), (1,)), ((), ())),
        preferred_element_type=jnp.float32)
    n_ref[...] = jnp.sum(xi * xi, axis=1, keepdims=True)


def _matmul_norms(x):
    return pl.pallas_call(
        _mm_body,
        grid=(NB,),
        in_specs=[pl.BlockSpec((N_ROWS, D), lambda i: (0, 0))],
        out_specs=[
            pl.BlockSpec((BM, N_ROWS), lambda i: (i, 0)),
            pl.BlockSpec((BM, 1), lambda i: (i, 0)),
        ],
        out_shape=[
            jax.ShapeDtypeStruct((N_ROWS, N_ROWS), jnp.float32),
            jax.ShapeDtypeStruct((N_ROWS, 1), jnp.float32),
        ],
        scratch_shapes=[pltpu.VMEM((N_ROWS, D), jnp.bfloat16)],
    )(x)


def _sc_diff_body(g_hbm, n_hbm, t_hbm, out_hbm,
                  n_v, tv_v, iv_v, jv_v, kv_v, fij_v, fik_v,
                  gij_v, gik_v, diff_v, sem):
    cid = lax.axis_index("c")
    sid = lax.axis_index("s")
    wid = sid * NC + cid
    base = wid * TPT

    pltpu.sync_copy(n_hbm, n_v)
    pltpu.sync_copy(t_hbm.at[pl.ds(base * 3, TPT * 3)], tv_v)

    lanes3 = lax.iota(jnp.int32, L) * 3

    def build_body(g, carry):
        sl = pl.ds(g * L, L)
        b = lanes3 + g * (3 * L)
        iv = plsc.load_gather(tv_v, [b])
        jv = plsc.load_gather(tv_v, [b + 1])
        kv = plsc.load_gather(tv_v, [b + 2])
        iv_v[sl] = iv
        jv_v[sl] = jv
        kv_v[sl] = kv
        rowbase = lax.shift_left(iv, 12)
        fij_v[sl] = rowbase + jv
        fik_v[sl] = rowbase + kv
        return carry

    lax.fori_loop(0, GRP, build_body, 0)

    copies = []
    for c in range(TPT // IDX_CH):
        sl = pl.ds(c * IDX_CH, IDX_CH)
        copies.append(pltpu.async_copy(
            g_hbm.at[fij_v.at[sl]], gij_v.at[sl], sem))
        copies.append(pltpu.async_copy(
            g_hbm.at[fik_v.at[sl]], gik_v.at[sl], sem))
    for cp in copies:
        cp.wait()

    def diff_body(g, carry):
        sl = pl.ds(g * L, L)
        ni = plsc.load_gather(n_v, [iv_v[sl]])
        nj = plsc.load_gather(n_v, [jv_v[sl]])
        nk = plsc.load_gather(n_v, [kv_v[sl]])
        gij = gij_v[sl]
        gik = gik_v[sl]
        dij = jnp.maximum(ni + nj - (gij + gij), 0.0)
        dik = jnp.maximum(ni + nk - (gik + gik), 0.0)
        diff_v[sl] = dij - dik
        return carry

    lax.fori_loop(0, GRP, diff_body, 0)
    pltpu.sync_copy(diff_v, out_hbm.at[pl.ds(base, TPT)])


_sc_diffs = functools.partial(
    pl.kernel,
    out_type=jax.ShapeDtypeStruct((N_TRIP,), jnp.float32),
    mesh=plsc.VectorSubcoreMesh(
        core_axis_name="c", subcore_axis_name="s",
        num_cores=NC, num_subcores=NS),
    compiler_params=pltpu.CompilerParams(needs_layout_passes=False),
    scratch_types=[
        pltpu.VMEM((N_ROWS,), jnp.float32),
        pltpu.VMEM((TPT * 3,), jnp.int32),
        pltpu.VMEM((TPT,), jnp.int32),
        pltpu.VMEM((TPT,), jnp.int32),
        pltpu.VMEM((TPT,), jnp.int32),
        pltpu.VMEM((TPT,), jnp.int32),
        pltpu.VMEM((TPT,), jnp.int32),
        pltpu.VMEM((TPT,), jnp.float32),
        pltpu.VMEM((TPT,), jnp.float32),
        pltpu.VMEM((TPT,), jnp.float32),
        pltpu.SemaphoreType.DMA,
    ],
)(_sc_diff_body)


def _tc_loss_body(d_ref, o_ref):
    per = jnp.log(1.0 + jnp.exp(d_ref[...]))
    o_ref[...] = jnp.reshape(jnp.sum(per) / float(N_TRIP), (1, 1))


def kernel(x, triplets):
    t_flat = triplets.astype(jnp.int32).reshape(N_TRIP * 3)
    g, n = _matmul_norms(x)
    diffs = _sc_diffs(g.reshape(N_ROWS * N_ROWS), n.reshape(N_ROWS),
                      t_flat)
    loss = pl.pallas_call(
        _tc_loss_body,
        out_shape=jax.ShapeDtypeStruct((1, 1), jnp.float32),
    )(diffs.reshape(128, 128))
    return loss.reshape(1)


# speedup vs baseline: 1.3553x; 1.1338x over previous
import functools

import jax
import jax.numpy as jnp
from jax import lax
from jax.experimental import pallas as pl
from jax.experimental.pallas import tpu as pltpu
from jax.experimental.pallas import tpu_sc as plsc

NC = 2
NS = 16
NW = NC * NS
L = 16

N_ROWS = 4096
D = 512
N_TRIP = 16384
TPT = N_TRIP // NW
GRP = TPT // L
BM = 512
NB = N_ROWS // BM
IDX_CH = 128


def _mm_body(x_ref, g_ref, n_ref, xb_scr):
    i = pl.program_id(0)

    @pl.when(i == 0)
    def _():
        xb_scr[...] = x_ref[...].astype(jnp.bfloat16)

    xi = x_ref[pl.ds(i * BM, BM), :]
    res = lax.dot_general(
        xb_scr[pl.ds(i * BM, BM), :], xb_scr[...],
        (((1,), (1,)), ((), ())),
        preferred_element_type=jnp.float32)
    for m in range(N_ROWS // 128):
        g_ref[:, m, :] = res[:, m * 128:(m + 1) * 128]
    n_ref[...] = jnp.sum(xi * xi, axis=1, keepdims=True)


def _matmul_norms(x):
    return pl.pallas_call(
        _mm_body,
        grid=(NB,),
        in_specs=[pl.BlockSpec((N_ROWS, D), lambda i: (0, 0))],
        out_specs=[
            pl.BlockSpec((BM, N_ROWS // 128, 128), lambda i: (i, 0, 0)),
            pl.BlockSpec((BM, 1), lambda i: (i, 0)),
        ],
        out_shape=[
            jax.ShapeDtypeStruct((N_ROWS, N_ROWS // 128, 128), jnp.float32),
            jax.ShapeDtypeStruct((N_ROWS, 1), jnp.float32),
        ],
        scratch_shapes=[pltpu.VMEM((N_ROWS, D), jnp.bfloat16)],
    )(x)


def _sc_diff_body(g_hbm, n_hbm, t_hbm, out_hbm,
                  n_v, tv_v, iv_v, jv_v, kv_v, fij_v, fik_v,
                  gij_v, gik_v, diff_v, sem):
    cid = lax.axis_index("c")
    sid = lax.axis_index("s")
    wid = sid * NC + cid
    base = wid * TPT

    pltpu.sync_copy(n_hbm, n_v)
    pltpu.sync_copy(t_hbm.at[pl.ds(base * 3, TPT * 3)], tv_v)

    lanes3 = lax.iota(jnp.int32, L) * 3

    def build_body(g, carry):
        sl = pl.ds(g * L, L)
        b = lanes3 + g * (3 * L)
        iv = plsc.load_gather(tv_v, [b])
        jv = plsc.load_gather(tv_v, [b + 1])
        kv = plsc.load_gather(tv_v, [b + 2])
        iv_v[sl] = iv
        jv_v[sl] = jv
        kv_v[sl] = kv
        rowbase = lax.shift_left(iv, 12)
        fij_v[sl] = rowbase + jv
        fik_v[sl] = rowbase + kv
        return carry

    lax.fori_loop(0, GRP, build_body, 0)

    copies = []
    for c in range(TPT // IDX_CH):
        sl = pl.ds(c * IDX_CH, IDX_CH)
        copies.append(pltpu.async_copy(
            g_hbm.at[fij_v.at[sl]], gij_v.at[sl], sem))
        copies.append(pltpu.async_copy(
            g_hbm.at[fik_v.at[sl]], gik_v.at[sl], sem))
    for cp in copies:
        cp.wait()

    def diff_body(g, carry):
        sl = pl.ds(g * L, L)
        ni = plsc.load_gather(n_v, [iv_v[sl]])
        nj = plsc.load_gather(n_v, [jv_v[sl]])
        nk = plsc.load_gather(n_v, [kv_v[sl]])
        gij = gij_v[sl]
        gik = gik_v[sl]
        dij = jnp.maximum(ni + nj - (gij + gij), 0.0)
        dik = jnp.maximum(ni + nk - (gik + gik), 0.0)
        diff_v[sl] = dij - dik
        return carry

    lax.fori_loop(0, GRP, diff_body, 0)
    pltpu.sync_copy(diff_v, out_hbm.at[pl.ds(base, TPT)])


_sc_diffs = functools.partial(
    pl.kernel,
    out_type=jax.ShapeDtypeStruct((N_TRIP,), jnp.float32),
    mesh=plsc.VectorSubcoreMesh(
        core_axis_name="c", subcore_axis_name="s",
        num_cores=NC, num_subcores=NS),
    compiler_params=pltpu.CompilerParams(needs_layout_passes=False),
    scratch_types=[
        pltpu.VMEM((N_ROWS,), jnp.float32),
        pltpu.VMEM((TPT * 3,), jnp.int32),
        pltpu.VMEM((TPT,), jnp.int32),
        pltpu.VMEM((TPT,), jnp.int32),
        pltpu.VMEM((TPT,), jnp.int32),
        pltpu.VMEM((TPT,), jnp.int32),
        pltpu.VMEM((TPT,), jnp.int32),
        pltpu.VMEM((TPT,), jnp.float32),
        pltpu.VMEM((TPT,), jnp.float32),
        pltpu.VMEM((TPT,), jnp.float32),
        pltpu.SemaphoreType.DMA,
    ],
)(_sc_diff_body)


def _tc_loss_body(d_ref, o_ref):
    per = jnp.log(1.0 + jnp.exp(d_ref[...]))
    o_ref[...] = jnp.reshape(jnp.sum(per) / float(N_TRIP), (1, 1))


def kernel(x, triplets):
    t_flat = triplets.astype(jnp.int32).reshape(N_TRIP * 3)
    g, n = _matmul_norms(x)
    diffs = _sc_diffs(g.reshape(N_ROWS * N_ROWS), n.reshape(N_ROWS),
                      t_flat)
    loss = pl.pallas_call(
        _tc_loss_body,
        out_shape=jax.ShapeDtypeStruct((1, 1), jnp.float32),
    )(diffs.reshape(128, 128))
    return loss.reshape(1)


# speedup vs baseline: 1.7183x; 1.2678x over previous
import functools

import jax
import jax.numpy as jnp
from jax import lax
from jax.experimental import pallas as pl
from jax.experimental.pallas import tpu as pltpu
from jax.experimental.pallas import tpu_sc as plsc

NC = 2
NS = 16
NW = NC * NS
L = 16

N_ROWS = 4096
D = 512
N_TRIP = 16384
TPT = N_TRIP // NW
GRP = TPT // L
BM = 512
NB = N_ROWS // BM
IDX_CH = 128


def _mm_body(x_ref, g_ref, n_ref, xb_scr):
    i = pl.program_id(0)

    @pl.when(i == 0)
    def _():
        xb_scr[...] = x_ref[...].astype(jnp.bfloat16)

    xi = x_ref[pl.ds(i * BM, BM), :]
    res = lax.dot_general(
        xb_scr[pl.ds(i * BM, BM), :], xb_scr[...],
        (((1,), (1,)), ((), ())),
        preferred_element_type=jnp.float32)
    g_ref[...] = res.reshape(BM, N_ROWS // 128, 128)
    n_ref[...] = jnp.sum(xi * xi, axis=1, keepdims=True)


def _matmul_norms(x):
    return pl.pallas_call(
        _mm_body,
        grid=(NB,),
        in_specs=[pl.BlockSpec((N_ROWS, D), lambda i: (0, 0))],
        out_specs=[
            pl.BlockSpec((BM, N_ROWS // 128, 128), lambda i: (i, 0, 0)),
            pl.BlockSpec((BM, 1), lambda i: (i, 0)),
        ],
        out_shape=[
            jax.ShapeDtypeStruct((N_ROWS, N_ROWS // 128, 128), jnp.float32),
            jax.ShapeDtypeStruct((N_ROWS, 1), jnp.float32),
        ],
        scratch_shapes=[pltpu.VMEM((N_ROWS, D), jnp.bfloat16)],
    )(x)


def _sc_diff_body(g_hbm, n_hbm, t_hbm, out_hbm,
                  n_v, tv_v, iv_v, jv_v, kv_v, fij_v, fik_v,
                  gij_v, gik_v, diff_v, sem):
    cid = lax.axis_index("c")
    sid = lax.axis_index("s")
    wid = sid * NC + cid
    base = wid * TPT

    pltpu.sync_copy(n_hbm, n_v)
    pltpu.sync_copy(t_hbm.at[pl.ds(base * 3, TPT * 3)], tv_v)

    lanes3 = lax.iota(jnp.int32, L) * 3

    def build_body(g, carry):
        sl = pl.ds(g * L, L)
        b = lanes3 + g * (3 * L)
        iv = plsc.load_gather(tv_v, [b])
        jv = plsc.load_gather(tv_v, [b + 1])
        kv = plsc.load_gather(tv_v, [b + 2])
        iv_v[sl] = iv
        jv_v[sl] = jv
        kv_v[sl] = kv
        rowbase = lax.shift_left(iv, 12)
        fij_v[sl] = rowbase + jv
        fik_v[sl] = rowbase + kv
        return carry

    lax.fori_loop(0, GRP, build_body, 0)

    copies = []
    for c in range(TPT // IDX_CH):
        sl = pl.ds(c * IDX_CH, IDX_CH)
        copies.append(pltpu.async_copy(
            g_hbm.at[fij_v.at[sl]], gij_v.at[sl], sem))
        copies.append(pltpu.async_copy(
            g_hbm.at[fik_v.at[sl]], gik_v.at[sl], sem))
    for cp in copies:
        cp.wait()

    def diff_body(g, carry):
        sl = pl.ds(g * L, L)
        ni = plsc.load_gather(n_v, [iv_v[sl]])
        nj = plsc.load_gather(n_v, [jv_v[sl]])
        nk = plsc.load_gather(n_v, [kv_v[sl]])
        gij = gij_v[sl]
        gik = gik_v[sl]
        dij = jnp.maximum(ni + nj - (gij + gij), 0.0)
        dik = jnp.maximum(ni + nk - (gik + gik), 0.0)
        diff_v[sl] = dij - dik
        return carry

    lax.fori_loop(0, GRP, diff_body, 0)
    pltpu.sync_copy(diff_v, out_hbm.at[pl.ds(base, TPT)])


_sc_diffs = functools.partial(
    pl.kernel,
    out_type=jax.ShapeDtypeStruct((N_TRIP,), jnp.float32),
    mesh=plsc.VectorSubcoreMesh(
        core_axis_name="c", subcore_axis_name="s",
        num_cores=NC, num_subcores=NS),
    compiler_params=pltpu.CompilerParams(needs_layout_passes=False),
    scratch_types=[
        pltpu.VMEM((N_ROWS,), jnp.float32),
        pltpu.VMEM((TPT * 3,), jnp.int32),
        pltpu.VMEM((TPT,), jnp.int32),
        pltpu.VMEM((TPT,), jnp.int32),
        pltpu.VMEM((TPT,), jnp.int32),
        pltpu.VMEM((TPT,), jnp.int32),
        pltpu.VMEM((TPT,), jnp.int32),
        pltpu.VMEM((TPT,), jnp.float32),
        pltpu.VMEM((TPT,), jnp.float32),
        pltpu.VMEM((TPT,), jnp.float32),
        pltpu.SemaphoreType.DMA,
    ],
)(_sc_diff_body)


def _tc_loss_body(d_ref, o_ref):
    per = jnp.log(1.0 + jnp.exp(d_ref[...]))
    o_ref[...] = jnp.reshape(jnp.sum(per) / float(N_TRIP), (1, 1))


def kernel(x, triplets):
    t_flat = triplets.astype(jnp.int32).reshape(N_TRIP * 3)
    g, n = _matmul_norms(x)
    diffs = _sc_diffs(g.reshape(N_ROWS * N_ROWS), n.reshape(N_ROWS),
                      t_flat)
    loss = pl.pallas_call(
        _tc_loss_body,
        out_shape=jax.ShapeDtypeStruct((1, 1), jnp.float32),
    )(diffs.reshape(128, 128))
    return loss.reshape(1)


# speedup vs baseline: 1.7649x; 1.0271x over previous
import functools

import jax
import jax.numpy as jnp
from jax import lax
from jax.experimental import pallas as pl
from jax.experimental.pallas import tpu as pltpu
from jax.experimental.pallas import tpu_sc as plsc

NC = 2
NS = 16
NW = NC * NS
L = 16

N_ROWS = 4096
D = 512
N_TRIP = 16384
TPT = N_TRIP // NW
GRP = TPT // L
BM = 512
NB = N_ROWS // BM
IDX_CH = 128


def _mm_body(x_ref, g_ref, n_ref, xb_scr):
    i = pl.program_id(0)

    @pl.when(i == 0)
    def _():
        xf = x_ref[...]
        xb_scr[...] = xf.astype(jnp.bfloat16)
        n_ref[...] = jnp.sum(xf * xf, axis=1).reshape(N_ROWS // 128, 128)

    res = lax.dot_general(
        xb_scr[pl.ds(i * BM, BM), :], xb_scr[...],
        (((1,), (1,)), ((), ())),
        preferred_element_type=jnp.float32)
    g_ref[...] = res.reshape(BM, N_ROWS // 128, 128)


def _matmul_norms(x):
    return pl.pallas_call(
        _mm_body,
        grid=(NB,),
        in_specs=[pl.BlockSpec((N_ROWS, D), lambda i: (0, 0))],
        out_specs=[
            pl.BlockSpec((BM, N_ROWS // 128, 128), lambda i: (i, 0, 0)),
            pl.BlockSpec((N_ROWS // 128, 128), lambda i: (0, 0)),
        ],
        out_shape=[
            jax.ShapeDtypeStruct((N_ROWS, N_ROWS // 128, 128), jnp.float32),
            jax.ShapeDtypeStruct((N_ROWS // 128, 128), jnp.float32),
        ],
        scratch_shapes=[pltpu.VMEM((N_ROWS, D), jnp.bfloat16)],
    )(x)


def _sc_diff_body(g_hbm, n_hbm, t_hbm, out_hbm,
                  n_v, tv_v, iv_v, jv_v, kv_v, fij_v, fik_v,
                  gij_v, gik_v, diff_v, sem):
    cid = lax.axis_index("c")
    sid = lax.axis_index("s")
    wid = sid * NC + cid
    base = wid * TPT

    pltpu.sync_copy(n_hbm, n_v)
    pltpu.sync_copy(t_hbm.at[pl.ds(base * 3, TPT * 3)], tv_v)

    lanes3 = lax.iota(jnp.int32, L) * 3

    def build_body(g, carry):
        sl = pl.ds(g * L, L)
        b = lanes3 + g * (3 * L)
        iv = plsc.load_gather(tv_v, [b])
        jv = plsc.load_gather(tv_v, [b + 1])
        kv = plsc.load_gather(tv_v, [b + 2])
        iv_v[sl] = iv
        jv_v[sl] = jv
        kv_v[sl] = kv
        rowbase = lax.shift_left(iv, 12)
        fij_v[sl] = rowbase + jv
        fik_v[sl] = rowbase + kv
        return carry

    lax.fori_loop(0, GRP, build_body, 0)

    copies = []
    for c in range(TPT // IDX_CH):
        sl = pl.ds(c * IDX_CH, IDX_CH)
        copies.append(pltpu.async_copy(
            g_hbm.at[fij_v.at[sl]], gij_v.at[sl], sem))
        copies.append(pltpu.async_copy(
            g_hbm.at[fik_v.at[sl]], gik_v.at[sl], sem))
    for cp in copies:
        cp.wait()

    def norm_gather(idx):
        return plsc.load_gather(
            n_v, [lax.shift_right_logical(idx, 7), idx & 127])

    def diff_body(g, carry):
        sl = pl.ds(g * L, L)
        ni = norm_gather(iv_v[sl])
        nj = norm_gather(jv_v[sl])
        nk = norm_gather(kv_v[sl])
        gij = gij_v[sl]
        gik = gik_v[sl]
        dij = jnp.maximum(ni + nj - (gij + gij), 0.0)
        dik = jnp.maximum(ni + nk - (gik + gik), 0.0)
        diff_v[g // 8, pl.ds((g % 8) * L, L)] = dij - dik
        return carry

    lax.fori_loop(0, GRP, diff_body, 0)
    pltpu.sync_copy(diff_v,
                    out_hbm.at[pl.ds(wid * (TPT // 128), TPT // 128)])


_sc_diffs = functools.partial(
    pl.kernel,
    out_type=jax.ShapeDtypeStruct((128, 128), jnp.float32),
    mesh=plsc.VectorSubcoreMesh(
        core_axis_name="c", subcore_axis_name="s",
        num_cores=NC, num_subcores=NS),
    compiler_params=pltpu.CompilerParams(needs_layout_passes=False),
    scratch_types=[
        pltpu.VMEM((N_ROWS // 128, 128), jnp.float32),
        pltpu.VMEM((TPT * 3,), jnp.int32),
        pltpu.VMEM((TPT,), jnp.int32),
        pltpu.VMEM((TPT,), jnp.int32),
        pltpu.VMEM((TPT,), jnp.int32),
        pltpu.VMEM((TPT,), jnp.int32),
        pltpu.VMEM((TPT,), jnp.int32),
        pltpu.VMEM((TPT,), jnp.float32),
        pltpu.VMEM((TPT,), jnp.float32),
        pltpu.VMEM((TPT // 128, 128), jnp.float32),
        pltpu.SemaphoreType.DMA,
    ],
)(_sc_diff_body)


def _tc_loss_body(d_ref, o_ref):
    per = jnp.log(1.0 + jnp.exp(d_ref[...]))
    o_ref[...] = jnp.reshape(jnp.sum(per) / float(N_TRIP), (1, 1))


def kernel(x, triplets):
    t_flat = triplets.astype(jnp.int32).reshape(N_TRIP * 3)
    g, n = _matmul_norms(x)
    diffs = _sc_diffs(g.reshape(N_ROWS * N_ROWS), n, t_flat)
    loss = pl.pallas_call(
        _tc_loss_body,
        out_shape=jax.ShapeDtypeStruct((1, 1), jnp.float32),
    )(diffs)
    return loss.reshape(1)
